# baseline (speedup 1.0000x reference)
import jax
import jax.numpy as jnp
from jax import lax
from jax.experimental import pallas as pl
from jax.experimental.pallas import tpu as pltpu
from jax.experimental.pallas import tpu_sc as plsc

DIM = 64
CHUNK = 800
NBUF = 2


def _gather_body(idx_hbm, table_hbm, out_hbm, idx_all, rows0, rows1,
                 gsem0, gsem1, ssem0, ssem1):
    nc = 2
    wid = lax.axis_index("s") * nc + lax.axis_index("c")
    b_per_w = idx_hbm.shape[0] // 32
    base = wid * b_per_w
    n = b_per_w // CHUNK
    rows = (rows0, rows1)
    gsem = (gsem0, gsem1)
    ssem = (ssem0, ssem1)

    pltpu.sync_copy(idx_hbm.at[pl.ds(base, b_per_w)], idx_all)

    def idxs(g):
        return idx_all.at[pl.ds(g * CHUNK, CHUNK)]

    def out_at(g):
        return out_hbm.at[pl.ds(base + g * CHUNK, CHUNK)]

    def start_gather(g, b):
        pltpu.async_copy(table_hbm.at[idxs(g)], rows[b], gsem[b])

    def wait_gather(g, b):
        pltpu.make_async_copy(table_hbm.at[idxs(g)], rows[b], gsem[b]).wait()

    def start_store(g, b):
        pltpu.async_copy(rows[b], out_at(g), ssem[b])

    def wait_store(g, b):
        pltpu.make_async_copy(rows[b], out_at(g), ssem[b]).wait()

    start_gather(0, 0)
    start_gather(1, 1)

    @pl.loop(0, n // NBUF)
    def _(s):
        for b in range(NBUF):
            g = s * NBUF + b
            wait_gather(g, b)
            start_store(g, b)

            @pl.when(g + NBUF < n)
            def _():
                wait_store(g, b)
                start_gather(g + NBUF, b)

    for b in range(NBUF):
        wait_store(n - NBUF + b, b)


def kernel(x, weight):
    rows, cols = x.shape
    b = rows * cols
    xf = x.reshape(b)
    b_per_w = b // 32
    mesh = plsc.VectorSubcoreMesh(
        core_axis_name="c", subcore_axis_name="s", num_cores=2, num_subcores=16
    )
    out = pl.kernel(
        _gather_body,
        out_type=jax.ShapeDtypeStruct((b, DIM), jnp.float32),
        mesh=mesh,
        compiler_params=pltpu.CompilerParams(use_tc_tiling_on_sc=False),
        scratch_types=[
            pltpu.VMEM((b_per_w,), jnp.int32),
            pltpu.VMEM((CHUNK, DIM), jnp.float32),
            pltpu.VMEM((CHUNK, DIM), jnp.float32),
            pltpu.SemaphoreType.DMA,
            pltpu.SemaphoreType.DMA,
            pltpu.SemaphoreType.DMA,
            pltpu.SemaphoreType.DMA,
        ],
    )(xf, weight)
    return out.reshape(rows, cols, DIM)


# speedup vs baseline: 1.0007x; 1.0007x over previous
import jax
import jax.numpy as jnp
from jax import lax
from jax.experimental import pallas as pl
from jax.experimental.pallas import tpu as pltpu
from jax.experimental.pallas import tpu_sc as plsc

DIM = 64
CHUNK = 512
NBUF = 2


def _gather_body(idx_hbm, table_hbm, out_hbm, idx_all, rows0, rows1,
                 gsem0, gsem1, ssem0, ssem1):
    nc = 2
    wid = lax.axis_index("s") * nc + lax.axis_index("c")
    b_per_w = idx_hbm.shape[0] // 32
    base = wid * b_per_w
    n = b_per_w // CHUNK
    rows = (rows0, rows1)
    gsem = (gsem0, gsem1)
    ssem = (ssem0, ssem1)

    pltpu.sync_copy(idx_hbm.at[pl.ds(base, b_per_w)], idx_all)

    def idxs(g):
        return idx_all.at[pl.ds(g * CHUNK, CHUNK)]

    def out_at(g):
        return out_hbm.at[pl.ds(base + g * CHUNK, CHUNK)]

    def start_gather(g, b):
        pltpu.async_copy(table_hbm.at[idxs(g)], rows[b], gsem[b])

    def wait_gather(g, b):
        pltpu.make_async_copy(table_hbm.at[idxs(g)], rows[b], gsem[b]).wait()

    def start_store(g, b):
        pltpu.async_copy(rows[b], out_at(g), ssem[b])

    def wait_store(g, b):
        pltpu.make_async_copy(rows[b], out_at(g), ssem[b]).wait()

    start_gather(0, 0)
    start_gather(1, 1)

    @pl.loop(0, n // NBUF)
    def _(s):
        for b in range(NBUF):
            g = s * NBUF + b
            wait_gather(g, b)
            start_store(g, b)

            @pl.when(g + NBUF < n)
            def _():
                wait_store(g, b)
                start_gather(g + NBUF, b)

    for b in range(NBUF):
        wait_store(n - NBUF + b, b)


def kernel(x, weight):
    rows, cols = x.shape
    b = rows * cols
    xf = x.reshape(b)
    b_per_w = b // 32
    mesh = plsc.VectorSubcoreMesh(
        core_axis_name="c", subcore_axis_name="s", num_cores=2, num_subcores=16
    )
    out = pl.kernel(
        _gather_body,
        out_type=jax.ShapeDtypeStruct((b, DIM), jnp.float32),
        mesh=mesh,
        compiler_params=pltpu.CompilerParams(use_tc_tiling_on_sc=False),
        scratch_types=[
            pltpu.VMEM((b_per_w,), jnp.int32),
            pltpu.VMEM((CHUNK, DIM), jnp.float32),
            pltpu.VMEM((CHUNK, DIM), jnp.float32),
            pltpu.SemaphoreType.DMA,
            pltpu.SemaphoreType.DMA,
            pltpu.SemaphoreType.DMA,
            pltpu.SemaphoreType.DMA,
        ],
    )(xf, weight)
    return out.reshape(rows, cols, DIM)
